# Initial kernel scaffold; baseline (speedup 1.0000x reference)
import jax
import jax.numpy as jnp
from jax.experimental import pallas as pl


def kernel(ego_embeddings, A_in_index, A_in_values, A_in_plusI_index, A_in_plusI_values, W1, b1, W2, b2):
    raise NotImplementedError("write your pallas kernel here")


# speedup vs baseline: 5.0233x; 5.0233x over previous
import functools

import jax
import jax.numpy as jnp
from jax import lax
from jax.experimental import pallas as pl
from jax.experimental.pallas import tpu as pltpu
from jax.experimental.pallas import tpu_sc as plsc

N = 10000
NPAD = 10240
D = 128
NC = 2
NS = 16
NW = NC * NS
CH = 96
NBUF = 3
ROWS_PER_TILE = NPAD // NS
ZROWS = 64


def _sc_spmm_body(ego, packed3, valr, out,
                  packed_v, b0, b1, b2, c0, c1, c2, r0, r1, r2, v0, v1, v2,
                  acc, sem_g, sem_v, sem_s, nchunks):
    c = lax.axis_index("c")
    s = lax.axis_index("s")
    wid = c * NS + s
    bufs = (b0, b1, b2)
    colss = (c0, c1, c2)
    rowss = (r0, r1, r2)
    vbs = (v0, v1, v2)
    slab = (nchunks + 1) * CH

    def zrow(r, carry):
        for k in range(D // 16):
            b0[r, pl.ds(k * 16, 16)] = jnp.zeros((16,), jnp.float32)
        return carry
    lax.fori_loop(0, ZROWS, zrow, 0)
    base = s * ROWS_PER_TILE

    def zcopy(k, carry):
        pltpu.sync_copy(b0.at[pl.ds(0, ZROWS)],
                        acc.at[pl.ds(base + k * ZROWS, ZROWS)])
        return carry
    lax.fori_loop(0, ROWS_PER_TILE // ZROWS, zcopy, 0)
    plsc.subcore_barrier()

    pltpu.sync_copy(packed3.at[pl.ds(wid * slab, slab)], packed_v)

    def unpack(j, p):
        for g in range(CH // 16):
            pk = packed_v[pl.ds(j * CH + g * 16, 16)]
            colss[p][pl.ds(g * 16, 16)] = pk & 0xFFFF
            rowss[p][pl.ds(g * 16, 16)] = pk >> 16

    def prefetch(j, p):
        unpack(j, p)
        pltpu.async_copy(ego.at[colss[p]], bufs[p], sem_g)
        pltpu.async_copy(valr.at[pl.ds(wid * slab + j * CH, CH)], vbs[p],
                         sem_v)

    def gwait(p):
        pltpu.make_async_copy(ego.at[colss[p]], bufs[p], sem_g).wait()
        pltpu.make_async_copy(valr.at[pl.ds(0, CH)], vbs[p], sem_v).wait()

    def swait(p):
        pltpu.make_async_copy(bufs[p], acc.at[rowss[p]], sem_s).wait()

    def scale(j, p):
        def group(g, cc):
            vv = vbs[p][pl.ds(g * 16, 16)]
            for l in range(16):
                v = vv[l]
                i = g * 16 + l
                for k in range(D // 16):
                    bufs[p][i, pl.ds(k * 16, 16)] = (
                        bufs[p][i, pl.ds(k * 16, 16)] * v)
            return cc
        lax.fori_loop(0, CH // 16, group, 0)

    def step(j, p, do_swait):
        if do_swait:
            swait((p + 1) % NBUF)
        prefetch(j + 1, (p + 1) % NBUF)
        gwait(p)
        scale(j, p)
        pltpu.async_copy(bufs[p], acc.at[rowss[p]], sem_s, add=True)

    prefetch(0, 0)
    step(0, 0, False)
    step(1, 1, False)
    step(2, 2, True)

    def triple(t, carry):
        j = t * 3
        step(j, 0, True)
        step(j + 1, 1, True)
        step(j + 2, 2, True)
        return carry
    lax.fori_loop(1, nchunks // 3, triple, 0)

    gwait(nchunks % NBUF)
    swait((nchunks - 2) % NBUF)
    swait((nchunks - 1) % NBUF)
    plsc.subcore_barrier()

    def ocopy(k, carry):
        pltpu.sync_copy(acc.at[pl.ds(base + k * ZROWS, ZROWS)],
                        out.at[c, pl.ds(base + k * ZROWS, ZROWS)])
        return carry
    lax.fori_loop(0, ROWS_PER_TILE // ZROWS, ocopy, 0)


def _sc_spmm(ego, packed3, vals3, nchunks):
    mesh = plsc.VectorSubcoreMesh(core_axis_name="c", subcore_axis_name="s")
    body = functools.partial(_sc_spmm_body, nchunks=nchunks)
    return pl.kernel(
        body,
        out_type=jax.ShapeDtypeStruct((NC, NPAD, D), jnp.float32),
        mesh=mesh,
        scratch_types=[
            pltpu.VMEM(((nchunks + 1) * CH,), jnp.int32),
            pltpu.VMEM((CH, D), jnp.float32),
            pltpu.VMEM((CH, D), jnp.float32),
            pltpu.VMEM((CH, D), jnp.float32),
            pltpu.VMEM((CH,), jnp.int32),
            pltpu.VMEM((CH,), jnp.int32),
            pltpu.VMEM((CH,), jnp.int32),
            pltpu.VMEM((CH,), jnp.int32),
            pltpu.VMEM((CH,), jnp.int32),
            pltpu.VMEM((CH,), jnp.int32),
            pltpu.VMEM((CH,), jnp.float32),
            pltpu.VMEM((CH,), jnp.float32),
            pltpu.VMEM((CH,), jnp.float32),
            pltpu.VMEM_SHARED((NPAD, D), jnp.float32),
            pltpu.SemaphoreType.DMA,
            pltpu.SemaphoreType.DMA,
            pltpu.SemaphoreType.DMA,
        ],
    )(ego, packed3, vals3)


def _tc_dense_body(ego_ref, p0_ref, p1_ref, w1_ref, w2_ref, b1_ref, b2_ref,
                   out_ref):
    side = p0_ref[...] + p1_ref[...]
    e = ego_ref[...]
    dn = (((1,), (1,)), ((), ()))
    h1 = lax.dot_general(e + side, w1_ref[...], dn,
                         preferred_element_type=jnp.float32) + b1_ref[...]
    h2 = lax.dot_general(e * side, w2_ref[...], dn,
                         preferred_element_type=jnp.float32) + b2_ref[...]
    out_ref[...] = (jnp.where(h1 >= 0, h1, 0.01 * h1)
                    + jnp.where(h2 >= 0, h2, 0.01 * h2))


def _tc_dense(ego, p0, p1, w1, w2, b1, b2):
    blk = 1000
    grid = (N // blk,)
    row_spec = pl.BlockSpec((blk, D), lambda i: (i, 0))
    full_spec = pl.BlockSpec((D, D), lambda i: (0, 0))
    bias_spec = pl.BlockSpec((1, D), lambda i: (0, 0))
    return pl.pallas_call(
        _tc_dense_body,
        grid=grid,
        in_specs=[row_spec, row_spec, row_spec, full_spec, full_spec,
                  bias_spec, bias_spec],
        out_specs=row_spec,
        out_shape=jax.ShapeDtypeStruct((N, D), jnp.float32),
    )(ego, p0, p1, w1, w2, b1, b2)


def kernel(ego_embeddings, A_in_index, A_in_values, A_in_plusI_index,
           A_in_plusI_values, W1, b1, W2, b2):
    e = A_in_values.shape[0]
    nchunks = -(-e // (NW * CH))
    nchunks = 3 * (-(-nchunks // 3))
    e_pad = nchunks * NW * CH
    rows = A_in_index[0].astype(jnp.int32)
    cols = A_in_index[1].astype(jnp.int32)
    vals = A_in_values.astype(jnp.float32)
    packed = (rows << 16) | cols
    pad = e_pad - e
    if pad:
        packed = jnp.concatenate([packed, jnp.zeros((pad,), jnp.int32)])
        vals = jnp.concatenate([vals, jnp.zeros((pad,), jnp.float32)])
    zc_i = jnp.zeros((NW, CH), jnp.int32)
    zc_f = jnp.zeros((NW, CH), jnp.float32)
    packed = jnp.concatenate(
        [packed.reshape(NW, nchunks * CH), zc_i], axis=1).reshape(-1)
    vals = jnp.concatenate(
        [vals.reshape(NW, nchunks * CH), zc_f], axis=1).reshape(-1)
    partials = _sc_spmm(ego_embeddings, packed, vals, nchunks)
    return _tc_dense(ego_embeddings, partials[0], partials[1], W1, W2,
                     b1.reshape(1, D), b2.reshape(1, D))
